# baseline (speedup 1.0000x reference)
import functools

import jax
import jax.numpy as jnp
from jax import lax
from jax.experimental import pallas as pl
from jax.experimental.pallas import tpu as pltpu
from jax.experimental.pallas import tpu_sc as plsc

N = 10000
N_PAD = 10240
D = 128
DH = D // 2
E = 320000
E_PAD = 327680
NC = 2
NS = 16
C = 128
EPT = E_PAD // NS
NCHUNK = EPT // C
NB = 8
CPB = NCHUNK // NB
RPT = N_PAD // NS
ZR = 64
DEGW = 16
NBUF = 5

_MESH = plsc.VectorSubcoreMesh(
    core_axis_name="c", subcore_axis_name="s", num_cores=NC, num_subcores=NS)


def _sc_body(with_deg, *refs):
    if with_deg:
        (xr_hbm, src_hbm, dst_hbm, agg_out, deg_out,
         src_b, dst_b, rows_v, zero_v, gsem,
         ones_v, degz_v, acc_sh, deg_sh) = refs
    else:
        (xr_hbm, src_hbm, dst_hbm, agg_out,
         src_b, dst_b, rows_v, zero_v, gsem, acc_sh) = refs

    cid = lax.axis_index("c")
    sid = lax.axis_index("s")
    r0 = sid * RPT

    @pl.loop(0, ZR)
    def _(r):
        for cc in range(DH // 16):
            zero_v[r, pl.ds(cc * 16, 16)] = jnp.zeros((16,), jnp.float32)
        if with_deg:
            degz_v[r, :] = jnp.zeros((16,), jnp.float32)

    if with_deg:
        @pl.loop(0, C)
        def _(r):
            ones_v[r, :] = jnp.ones((16,), jnp.float32)

    for k in range(RPT // ZR):
        pltpu.sync_copy(zero_v, acc_sh.at[pl.ds(r0 + k * ZR, ZR)])
        if with_deg:
            pltpu.sync_copy(degz_v, deg_sh.at[pl.ds(r0 + k * ZR, ZR)])

    plsc.subcore_barrier()

    @pl.loop(0, NB)
    def _(b):
        pltpu.sync_copy(src_hbm.at[sid, b], src_b)
        pltpu.sync_copy(dst_hbm.at[sid, b], dst_b)

        @pl.loop(0, CPB * C // 16)
        def _(i):
            r = i // (C // 16)
            cc = i % (C // 16)
            v = src_b[r, pl.ds(cc * 16, 16)]
            src_b[r, pl.ds(cc * 16, 16)] = v * 2 + cid

        if with_deg:
            do_deg = (b < NB // 2) == (cid == 0)

        @pl.loop(0, NBUF)
        def _(k):
            pltpu.async_copy(xr_hbm.at[src_b.at[k]],
                             rows_v.at[pl.ds(k * C, C)], gsem.at[k])

        @pl.loop(0, CPB)
        def _(c):
            bi = c % NBUF
            rows_slice = rows_v.at[pl.ds(bi * C, C)]
            pltpu.make_async_copy(xr_hbm.at[src_b.at[c]], rows_slice,
                                  gsem.at[bi]).wait()
            pltpu.sync_copy(rows_slice, acc_sh.at[dst_b.at[c]], add=True)
            if with_deg:
                @pl.when(do_deg)
                def _():
                    pltpu.sync_copy(ones_v, deg_sh.at[dst_b.at[c]], add=True)
            jn = c + NBUF

            @pl.when(jn < CPB)
            def _():
                pltpu.async_copy(xr_hbm.at[src_b.at[jn]], rows_slice,
                                 gsem.at[bi])

    plsc.subcore_barrier()

    pltpu.sync_copy(acc_sh.at[pl.ds(r0, RPT)], agg_out.at[cid, pl.ds(r0, RPT)])
    if with_deg:
        pltpu.sync_copy(deg_sh.at[pl.ds(r0, RPT)],
                        deg_out.at[cid, pl.ds(r0, RPT)])


_sc_scratch = [
    pltpu.VMEM((CPB, C), jnp.int32),
    pltpu.VMEM((CPB, C), jnp.int32),
    pltpu.VMEM((NBUF * C, DH), jnp.float32),
    pltpu.VMEM((ZR, DH), jnp.float32),
    pltpu.SemaphoreType.DMA((NBUF,)),
]


def _make_sc(with_deg, interpret=False):
    if with_deg:
        out_type = [
            jax.ShapeDtypeStruct((NC, N_PAD, DH), jnp.float32),
            jax.ShapeDtypeStruct((NC, N_PAD, DEGW), jnp.float32),
        ]
        scratch = _sc_scratch + [
            pltpu.VMEM((C, DEGW), jnp.float32),
            pltpu.VMEM((ZR, DEGW), jnp.float32),
            pltpu.VMEM_SHARED((N_PAD, DH), jnp.float32),
            pltpu.VMEM_SHARED((N_PAD, DEGW), jnp.float32),
        ]
    else:
        out_type = jax.ShapeDtypeStruct((NC, N_PAD, DH), jnp.float32)
        scratch = _sc_scratch + [
            pltpu.VMEM_SHARED((N_PAD, DH), jnp.float32),
        ]
    return pl.kernel(
        functools.partial(_sc_body, with_deg),
        out_type=out_type,
        mesh=_MESH,
        compiler_params=pltpu.CompilerParams(use_tc_tiling_on_sc=False),
        scratch_types=scratch,
        interpret=interpret,
    )


_sc_agg_deg = _make_sc(True)
_sc_agg = _make_sc(False)

_TC_R = 400


def _tc_self_body(h_ref, wst_ref, bs_ref, bn_ref, out_ref):
    acc = jnp.dot(h_ref[...], wst_ref[...], preferred_element_type=jnp.float32)
    out_ref[...] = acc + bs_ref[...] + bn_ref[...]


def _tc_self(h, wst, bs, bn):
    return pl.pallas_call(
        _tc_self_body,
        grid=(N // _TC_R,),
        in_specs=[
            pl.BlockSpec((_TC_R, D), lambda i: (i, 0)),
            pl.BlockSpec((D, D), lambda i: (0, 0)),
            pl.BlockSpec((1, D), lambda i: (0, 0)),
            pl.BlockSpec((1, D), lambda i: (0, 0)),
        ],
        out_specs=pl.BlockSpec((_TC_R, D), lambda i: (i, 0)),
        out_shape=jax.ShapeDtypeStruct((N, D), jnp.float32),
    )(h, wst, bs, bn)


def _tc_combine_body(s_ref, p_ref, d_ref, wnt_ref, out_ref):
    d0 = d_ref[0]
    d1 = d_ref[1]
    deg = d0[:, :1] + d1[:, :1]
    inv = 1.0 / jnp.maximum(deg, 1.0)
    neigh = jnp.concatenate([p_ref[0], p_ref[1]], axis=1) * inv
    acc = s_ref[...] + jnp.dot(neigh, wnt_ref[...],
                               preferred_element_type=jnp.float32)
    out_ref[...] = jnp.maximum(acc, 0.0)


def _tc_combine(s, p, d, wnt):
    return pl.pallas_call(
        _tc_combine_body,
        grid=(N // _TC_R,),
        in_specs=[
            pl.BlockSpec((_TC_R, D), lambda i: (i, 0)),
            pl.BlockSpec((NC, _TC_R, DH), lambda i: (0, i, 0)),
            pl.BlockSpec((NC, _TC_R, DEGW), lambda i: (0, i, 0)),
            pl.BlockSpec((D, D), lambda i: (0, 0)),
        ],
        out_specs=pl.BlockSpec((_TC_R, D), lambda i: (i, 0)),
        out_shape=jax.ShapeDtypeStruct((N, D), jnp.float32),
    )(s, p, d, wnt)


def kernel(in_feat, edge_index, W_self1, b_self1, W_neigh1, b_neigh1,
           W_self2, b_self2, W_neigh2, b_neigh2):
    ei = edge_index.astype(jnp.int32)
    npad = E_PAD - E
    pad = jnp.stack([
        jnp.zeros((npad,), jnp.int32),
        N + (jnp.arange(npad, dtype=jnp.int32) % (N_PAD - N)),
    ])
    ei = jnp.concatenate([ei, pad], axis=1)
    src4 = ei[0].reshape(NS, NB, CPB, C)
    dst4 = ei[1].reshape(NS, NB, CPB, C)

    xr1 = in_feat.reshape(2 * N, DH)
    agg_p, deg_p = _sc_agg_deg(xr1, src4, dst4)
    s1 = _tc_self(in_feat, W_self1.T, b_self1[None, :], b_neigh1[None, :])
    h1 = _tc_combine(s1, agg_p, deg_p, W_neigh1.T)

    xr2 = h1.reshape(2 * N, DH)
    agg2_p = _sc_agg(xr2, src4, dst4)
    s2 = _tc_self(h1, W_self2.T, b_self2[None, :], b_neigh2[None, :])
    h2 = _tc_combine(s2, agg2_p, deg_p, W_neigh2.T)
    return h2


# speedup vs baseline: 2.9267x; 2.9267x over previous
import functools

import jax
import jax.numpy as jnp
from jax import lax
from jax.experimental import pallas as pl
from jax.experimental.pallas import tpu as pltpu
from jax.experimental.pallas import tpu_sc as plsc

N = 10000
N_PAD = 10240
D = 128
DH = D // 2
E = 320000
NC = 2
NS = 16
C = 80
EPT = E // NS
NCHUNK = EPT // C
NB = 2
CPB = NCHUNK // NB
RPT = N_PAD // NS
ZR = 64
DEGW = 16
NBUF = 5

_MESH = plsc.VectorSubcoreMesh(
    core_axis_name="c", subcore_axis_name="s", num_cores=NC, num_subcores=NS)


def _sc_body(with_deg, *refs):
    if with_deg:
        (xr_hbm, src_hbm, dst_hbm, agg_out, deg_out,
         src_b, dst_b, rows_v, zero_v, gsem,
         ones_v, degz_v, acc_sh, deg_sh) = refs
    else:
        (xr_hbm, src_hbm, dst_hbm, agg_out,
         src_b, dst_b, rows_v, zero_v, gsem, acc_sh) = refs

    cid = lax.axis_index("c")
    sid = lax.axis_index("s")
    r0 = sid * RPT

    @pl.loop(0, ZR)
    def _(r):
        for cc in range(DH // 16):
            zero_v[r, pl.ds(cc * 16, 16)] = jnp.zeros((16,), jnp.float32)
        if with_deg:
            degz_v[r, :] = jnp.zeros((16,), jnp.float32)

    if with_deg:
        @pl.loop(0, C)
        def _(r):
            ones_v[r, :] = jnp.ones((16,), jnp.float32)

    for k in range(RPT // ZR):
        pltpu.sync_copy(zero_v, acc_sh.at[pl.ds(r0 + k * ZR, ZR)])
        if with_deg:
            pltpu.sync_copy(degz_v, deg_sh.at[pl.ds(r0 + k * ZR, ZR)])

    plsc.subcore_barrier()

    @pl.loop(0, NB)
    def _(b):
        pltpu.sync_copy(src_hbm.at[sid, b], src_b)
        pltpu.sync_copy(dst_hbm.at[sid, b], dst_b)

        @pl.loop(0, CPB * C // 16)
        def _(i):
            r = i // (C // 16)
            cc = i % (C // 16)
            v = src_b[r, pl.ds(cc * 16, 16)]
            src_b[r, pl.ds(cc * 16, 16)] = v * 2 + cid

        if with_deg:
            do_deg = (b < NB // 2) == (cid == 0)

        @pl.loop(0, NBUF)
        def _(k):
            pltpu.async_copy(xr_hbm.at[src_b.at[k]],
                             rows_v.at[pl.ds(k * C, C)], gsem.at[k])

        @pl.loop(0, CPB)
        def _(c):
            bi = c % NBUF
            rows_slice = rows_v.at[pl.ds(bi * C, C)]
            pltpu.make_async_copy(xr_hbm.at[src_b.at[c]], rows_slice,
                                  gsem.at[bi]).wait()
            pltpu.sync_copy(rows_slice, acc_sh.at[dst_b.at[c]], add=True)
            if with_deg:
                @pl.when(do_deg)
                def _():
                    pltpu.sync_copy(ones_v, deg_sh.at[dst_b.at[c]], add=True)
            jn = c + NBUF

            @pl.when(jn < CPB)
            def _():
                pltpu.async_copy(xr_hbm.at[src_b.at[jn]], rows_slice,
                                 gsem.at[bi])

    plsc.subcore_barrier()

    pltpu.sync_copy(acc_sh.at[pl.ds(r0, RPT)], agg_out.at[cid, pl.ds(r0, RPT)])
    if with_deg:
        pltpu.sync_copy(deg_sh.at[pl.ds(r0, RPT)],
                        deg_out.at[cid, pl.ds(r0, RPT)])


_sc_scratch = [
    pltpu.VMEM((CPB, C), jnp.int32),
    pltpu.VMEM((CPB, C), jnp.int32),
    pltpu.VMEM((NBUF * C, DH), jnp.float32),
    pltpu.VMEM((ZR, DH), jnp.float32),
    pltpu.SemaphoreType.DMA((NBUF,)),
]


def _make_sc(with_deg, interpret=False):
    if with_deg:
        out_type = [
            jax.ShapeDtypeStruct((NC, N_PAD, DH), jnp.float32),
            jax.ShapeDtypeStruct((NC, N_PAD, DEGW), jnp.float32),
        ]
        scratch = _sc_scratch + [
            pltpu.VMEM((C, DEGW), jnp.float32),
            pltpu.VMEM((ZR, DEGW), jnp.float32),
            pltpu.VMEM_SHARED((N_PAD, DH), jnp.float32),
            pltpu.VMEM_SHARED((N_PAD, DEGW), jnp.float32),
        ]
    else:
        out_type = jax.ShapeDtypeStruct((NC, N_PAD, DH), jnp.float32)
        scratch = _sc_scratch + [
            pltpu.VMEM_SHARED((N_PAD, DH), jnp.float32),
        ]
    return pl.kernel(
        functools.partial(_sc_body, with_deg),
        out_type=out_type,
        mesh=_MESH,
        compiler_params=pltpu.CompilerParams(use_tc_tiling_on_sc=False),
        scratch_types=scratch,
        interpret=interpret,
    )


_sc_agg_deg = _make_sc(True)
_sc_agg = _make_sc(False)

_TC_R = 400


def _tc_self_body(h_ref, wst_ref, bs_ref, bn_ref, out_ref):
    acc = jnp.dot(h_ref[...], wst_ref[...], preferred_element_type=jnp.float32)
    out_ref[...] = acc + bs_ref[...] + bn_ref[...]


def _tc_self(h, wst, bs, bn):
    return pl.pallas_call(
        _tc_self_body,
        grid=(N // _TC_R,),
        in_specs=[
            pl.BlockSpec((_TC_R, D), lambda i: (i, 0)),
            pl.BlockSpec((D, D), lambda i: (0, 0)),
            pl.BlockSpec((1, D), lambda i: (0, 0)),
            pl.BlockSpec((1, D), lambda i: (0, 0)),
        ],
        out_specs=pl.BlockSpec((_TC_R, D), lambda i: (i, 0)),
        out_shape=jax.ShapeDtypeStruct((N, D), jnp.float32),
    )(h, wst, bs, bn)


def _tc_combine_body(s_ref, p_ref, d_ref, wnt_ref, out_ref):
    d0 = d_ref[0]
    d1 = d_ref[1]
    deg = d0[:, :1] + d1[:, :1]
    inv = 1.0 / jnp.maximum(deg, 1.0)
    neigh = jnp.concatenate([p_ref[0], p_ref[1]], axis=1) * inv
    acc = s_ref[...] + jnp.dot(neigh, wnt_ref[...],
                               preferred_element_type=jnp.float32)
    out_ref[...] = jnp.maximum(acc, 0.0)


def _tc_combine(s, p, d, wnt):
    return pl.pallas_call(
        _tc_combine_body,
        grid=(N // _TC_R,),
        in_specs=[
            pl.BlockSpec((_TC_R, D), lambda i: (i, 0)),
            pl.BlockSpec((NC, _TC_R, DH), lambda i: (0, i, 0)),
            pl.BlockSpec((NC, _TC_R, DEGW), lambda i: (0, i, 0)),
            pl.BlockSpec((D, D), lambda i: (0, 0)),
        ],
        out_specs=pl.BlockSpec((_TC_R, D), lambda i: (i, 0)),
        out_shape=jax.ShapeDtypeStruct((N, D), jnp.float32),
    )(s, p, d, wnt)


def kernel(in_feat, edge_index, W_self1, b_self1, W_neigh1, b_neigh1,
           W_self2, b_self2, W_neigh2, b_neigh2):
    ei = edge_index.astype(jnp.int32)
    src4 = ei[0].reshape(NS, NB, CPB, C)
    dst4 = ei[1].reshape(NS, NB, CPB, C)

    xr1 = in_feat.reshape(2 * N, DH)
    agg_p, deg_p = _sc_agg_deg(xr1, src4, dst4)
    s1 = _tc_self(in_feat, W_self1.T, b_self1[None, :], b_neigh1[None, :])
    h1 = _tc_combine(s1, agg_p, deg_p, W_neigh1.T)

    xr2 = h1.reshape(2 * N, DH)
    agg2_p = _sc_agg(xr2, src4, dst4)
    s2 = _tc_self(h1, W_self2.T, b_self2[None, :], b_neigh2[None, :])
    h2 = _tc_combine(s2, agg2_p, deg_p, W_neigh2.T)
    return h2
